# Initial kernel scaffold; baseline (speedup 1.0000x reference)
import jax
import jax.numpy as jnp
from jax.experimental import pallas as pl


def kernel(pos, f_in, edge_src, edge_dst, batch_index, W1, W2):
    raise NotImplementedError("write your pallas kernel here")


# speedup vs baseline: 2.5971x; 2.5971x over previous
import functools

import jax
import jax.numpy as jnp
import numpy as np
from jax import lax
from jax.experimental import pallas as pl
from jax.experimental.pallas import tpu as pltpu
from jax.experimental.pallas import tpu_sc as plsc

N = 10000
E = 160000
D = 16
NB = 20
HID = 20
RADIUS = 5.0
NGRAPH = 8

NW = 32
CHUNK = 128
NCHUNK = 40
EPT = NCHUNK * CHUNK
E_PAD = NW * EPT
QTR = EPT // 4
NROW = N + 8
ZROWS = N // 16

_SUS_C = 1.14136 * float(np.exp(2.0))


def _sus(x):
    xp = jnp.where(x > 0.0, x, 1.0)
    return jnp.where(x > 0.0, jnp.exp(-1.0 / xp), 0.0)


_sc_mesh = plsc.VectorSubcoreMesh(core_axis_name="c", subcore_axis_name="s")


@functools.partial(
    pl.kernel,
    out_type=(
        jax.ShapeDtypeStruct((E_PAD, 2 * D), jnp.float32),
        jax.ShapeDtypeStruct((E_PAD, D), jnp.float32),
    ),
    mesh=_sc_mesh,
    scratch_types=[
        pltpu.VMEM((EPT,), jnp.int32),
        pltpu.VMEM((EPT,), jnp.int32),
        pltpu.VMEM((QTR, 2 * D), jnp.float32),
        pltpu.VMEM((QTR, D), jnp.float32),
        pltpu.SemaphoreType.DMA,
    ],
    compiler_params=pltpu.CompilerParams(use_tc_tiling_on_sc=False),
)
def _gather_sc(srctab_hbm, pos16_hbm, src_hbm, dst_hbm,
               sg_hbm, dg_hbm,
               sidx, didx, sbuf, dbuf, sem):
    wid = lax.axis_index("s") * 2 + lax.axis_index("c")
    base = wid * EPT
    pltpu.sync_copy(src_hbm.at[pl.ds(base, EPT)], sidx)
    pltpu.sync_copy(dst_hbm.at[pl.ds(base, EPT)], didx)
    for q in range(4):
        off = q * QTR
        a = pltpu.async_copy(srctab_hbm.at[sidx.at[pl.ds(off, QTR)]], sbuf, sem)
        b = pltpu.async_copy(pos16_hbm.at[didx.at[pl.ds(off, QTR)]], dbuf, sem)
        a.wait()
        b.wait()
        pltpu.sync_copy(sbuf, sg_hbm.at[pl.ds(base + off, QTR)])
        pltpu.sync_copy(dbuf, dg_hbm.at[pl.ds(base + off, QTR)])


EB = 1024


_HI = jax.lax.Precision.HIGHEST


def _edge_body(sg_ref, dg_ref, w1_ref, tm_ref, out_ref):
    sg = sg_ref[...]
    vec = dg_ref[..., :4] - sg[:, :4]
    xs = sg[:, D:]
    dsq = jnp.sum(vec * vec, axis=1, keepdims=True)
    d = jnp.sqrt(dsq + 1e-12)
    step = RADIUS / (NB + 1)
    basis = (lax.broadcasted_iota(jnp.int32, (EB, NB), 1).astype(jnp.float32)
             + 1.0) * step
    diff = (d - basis) / step
    emb = _SUS_C * _sus(diff + 1.0) * _sus(1.0 - diff) * np.sqrt(NB)
    w1s = w1_ref[...] * (1.0 / np.sqrt(NB))
    h = jnp.dot(emb, w1s, preferred_element_type=jnp.float32, precision=_HI)
    h = jnp.maximum(h, 0.0) * np.sqrt(2.0)
    jR = lax.broadcasted_iota(jnp.int32, (HID, HID * D), 1)
    kR = lax.broadcasted_iota(jnp.int32, (HID, HID * D), 0)
    R = (kR == jR // D).astype(jnp.float32)
    jS = lax.broadcasted_iota(jnp.int32, (D, HID * D), 1)
    uS = lax.broadcasted_iota(jnp.int32, (D, HID * D), 0)
    S = (uS == jS % D).astype(jnp.float32)
    z = (jnp.dot(h, R, preferred_element_type=jnp.float32, precision=_HI)
         * jnp.dot(xs, S, preferred_element_type=jnp.float32, precision=_HI))
    tm = tm_ref[...] * (1.0 / np.sqrt(HID))
    out_ref[...] = jnp.dot(z, tm, preferred_element_type=jnp.float32,
                           precision=_HI) * (1.0 / np.sqrt(D))


def _edge_tc(sg, dg, w1, tm):
    grid = E_PAD // EB
    return pl.pallas_call(
        _edge_body,
        grid=(grid,),
        in_specs=[
            pl.BlockSpec((EB, 2 * D), lambda i: (i, 0)),
            pl.BlockSpec((EB, D), lambda i: (i, 0)),
            pl.BlockSpec((HID, HID), lambda i: (0, 0)),
            pl.BlockSpec((HID * D, D), lambda i: (0, 0)),
        ],
        out_specs=pl.BlockSpec((EB, D), lambda i: (i, 0)),
        out_shape=jax.ShapeDtypeStruct((E_PAD, D), jnp.float32),
    )(sg, dg, w1, tm)


@functools.partial(
    pl.kernel,
    out_type=jax.ShapeDtypeStruct((2, N, D), jnp.float32),
    mesh=plsc.VectorSubcoreMesh(core_axis_name="c", subcore_axis_name="s"),
    scratch_types=[
        pltpu.VMEM((NCHUNK, CHUNK), jnp.int32),
        pltpu.VMEM((CHUNK, D), jnp.float32),
        pltpu.VMEM_SHARED((NROW, D), jnp.float32),
    ],
    compiler_params=pltpu.CompilerParams(use_tc_tiling_on_sc=False),
)
def _scatter_sc(fe_hbm, dst3_hbm, zeros_hbm, out_hbm, idxv, fbuf, shared):
    cid = lax.axis_index("c")
    sid = lax.axis_index("s")
    wid = sid * 2 + cid
    pltpu.sync_copy(zeros_hbm.at[pl.ds(sid * ZROWS, ZROWS)],
                    shared.at[pl.ds(sid * ZROWS, ZROWS)])
    pltpu.sync_copy(dst3_hbm.at[wid], idxv)
    plsc.subcore_barrier()

    def body(j, carry):
        pltpu.sync_copy(fe_hbm.at[pl.ds(wid * EPT + j * CHUNK, CHUNK)], fbuf)
        pltpu.sync_copy(fbuf, shared.at[idxv.at[j]], add=True)
        return carry

    lax.fori_loop(0, NCHUNK, body, 0)
    plsc.subcore_barrier()
    pltpu.sync_copy(shared.at[pl.ds(sid * ZROWS, ZROWS)],
                    out_hbm.at[cid, pl.ds(sid * ZROWS, ZROWS)])


def _final_body(p_ref, b_ref, src_ref, fout_ref, nn_ref):
    b = b_ref[...]
    src = src_ref[...]
    starts = [jnp.int32(0)]
    for g in range(1, NGRAPH):
        starts.append(jnp.sum((b < g).astype(jnp.int32)))
    starts.append(jnp.int32(N))
    cnts = [jnp.float32(0.0)]
    for g in range(1, NGRAPH):
        cnts.append(jnp.sum((src < starts[g]).astype(jnp.float32)))
    cnts.append(jnp.float32(E))
    nn = []
    for g in range(NGRAPH):
        n_edges = cnts[g + 1] - cnts[g]
        n_res = (starts[g + 1] - starts[g]).astype(jnp.float32)
        nn.append(n_edges / n_res)
    lane = lax.broadcasted_iota(jnp.int32, (1, NGRAPH), 1)
    nn_row = jnp.zeros((1, NGRAPH), jnp.float32)
    for g in range(NGRAPH):
        nn_row = nn_row + nn[g] * (lane == g).astype(jnp.float32)
    nn_ref[...] = nn_row
    scale = jnp.zeros((N, 1), jnp.float32)
    for g in range(NGRAPH):
        scale = scale + nn[g] * (b == g).astype(jnp.float32)
    fout_ref[...] = (p_ref[0] + p_ref[1]) / jnp.sqrt(scale)


def _final_tc(partials, batch2d, src2d):
    return pl.pallas_call(
        _final_body,
        in_specs=[
            pl.BlockSpec(partials.shape, lambda: (0, 0, 0)),
            pl.BlockSpec(batch2d.shape, lambda: (0, 0)),
            pl.BlockSpec(src2d.shape, lambda: (0, 0)),
        ],
        out_specs=[
            pl.BlockSpec((N, D), lambda: (0, 0)),
            pl.BlockSpec((1, NGRAPH), lambda: (0, 0)),
        ],
        out_shape=[
            jax.ShapeDtypeStruct((N, D), jnp.float32),
            jax.ShapeDtypeStruct((1, NGRAPH), jnp.float32),
        ],
    )(partials, batch2d, src2d)


def kernel(pos, f_in, edge_src, edge_dst, batch_index, W1, W2):
    srctab = jnp.concatenate(
        [pos, jnp.zeros((N, D - 3), jnp.float32), f_in], axis=1)
    pos16 = jnp.pad(pos, ((0, 8), (0, D - 3)))
    pad = E_PAD - E
    src_pad = jnp.concatenate([edge_src, jnp.zeros((pad,), jnp.int32)])
    dst_pad = jnp.concatenate([edge_dst, jnp.full((pad,), N, jnp.int32)])
    dst3 = dst_pad.reshape(NW, NCHUNK, CHUNK)
    zeros = jnp.zeros((N, D), jnp.float32)
    tm = W2.reshape(HID * D, D)

    sg, dg = _gather_sc(srctab, pos16, src_pad, dst_pad)
    fe = _edge_tc(sg, dg, W1, tm)
    partials = _scatter_sc(fe, dst3, zeros)
    f_out, nn2 = _final_tc(partials, batch_index.reshape(N, 1),
                           edge_src.reshape(E // 128, 128))
    return (f_out, edge_src, edge_dst, nn2.reshape(NGRAPH))
